# baseline (speedup 1.0000x reference)
import dataclasses
import functools

import jax
import jax.numpy as jnp
from jax import lax
from jax.experimental import pallas as pl
from jax.experimental.pallas import tpu as pltpu
from jax.experimental.pallas import tpu_sc as plsc

N_EMBED = 16
BATCH = 16384
NUM_CORES = 2
NUM_SUBCORES = 16
NUM_WORKERS = NUM_CORES * NUM_SUBCORES
B_PER_W = BATCH // NUM_WORKERS
LANES = 16
N_CHUNKS = 2
CHUNK = B_PER_W // N_CHUNKS
ROWS_PER_LINE = 128 // N_EMBED


def _sc_embed_dot(u_idx, i_idx, mu, bias, W_user, W_item):
    mesh = plsc.VectorSubcoreMesh(core_axis_name="c", subcore_axis_name="s")

    cp = pltpu.CompilerParams()
    if "needs_layout_passes" in pltpu.CompilerParams.__dataclass_fields__:
        cp = dataclasses.replace(cp, needs_layout_passes=False)

    @functools.partial(
        pl.kernel,
        compiler_params=cp,
        out_type=jax.ShapeDtypeStruct((BATCH,), jnp.float32),
        mesh=mesh,
        scratch_types=[
            pltpu.VMEM((B_PER_W,), jnp.int32),
            pltpu.VMEM((B_PER_W,), jnp.int32),
            pltpu.VMEM((CHUNK,), jnp.int32),
            pltpu.VMEM((CHUNK,), jnp.int32),
            pltpu.VMEM((CHUNK,), jnp.int32),
            pltpu.VMEM((CHUNK,), jnp.int32),
            pltpu.VMEM((CHUNK, 128), jnp.float32),
            pltpu.VMEM((CHUNK, 128), jnp.float32),
            pltpu.VMEM((B_PER_W,), jnp.float32),
            pltpu.VMEM((B_PER_W,), jnp.float32),
            pltpu.VMEM((B_PER_W,), jnp.float32),
            pltpu.SemaphoreType.DMA,
        ],
    )
    def k(u_hbm, i_hbm, mu_hbm, b_hbm, wu_hbm, wi_hbm, out_hbm,
          uidx_v, iidx_v, gu0_v, gu1_v, gi0_v, gi1_v, urows_v, irows_v,
          mu_v, b_v, out_v, sem):
        gu_refs = (gu0_v, gu1_v)
        gi_refs = (gi0_v, gi1_v)
        wid = lax.axis_index("s") * NUM_CORES + lax.axis_index("c")
        base = wid * B_PER_W
        sl = pl.ds(base, B_PER_W)

        pltpu.sync_copy(u_hbm.at[sl], uidx_v)
        pltpu.sync_copy(i_hbm.at[sl], iidx_v)

        for h in range(N_CHUNKS):
            @pl.loop(0, CHUNK, step=LANES)
            def _(c, h=h):
                g = h * CHUNK + c
                gu_refs[h][pl.ds(c, LANES)] = lax.shift_right_logical(
                    uidx_v[pl.ds(g, LANES)], 3)
                gi_refs[h][pl.ds(c, LANES)] = lax.shift_right_logical(
                    iidx_v[pl.ds(g, LANES)], 3)

        pltpu.sync_copy(mu_hbm.at[sl], mu_v)
        pltpu.sync_copy(b_hbm.at[sl], b_v)

        lane_iota = lax.iota(jnp.int32, LANES)

        for h in range(N_CHUNKS):
            cp_u = pltpu.async_copy(wu_hbm.at[gu_refs[h]], urows_v, sem)
            cp_i = pltpu.async_copy(wi_hbm.at[gi_refs[h]], irows_v, sem)
            cp_u.wait()
            cp_i.wait()

            @pl.loop(0, CHUNK, step=LANES)
            def _(c):
                g = h * CHUNK + c
                rows = lane_iota + c
                su = (uidx_v[pl.ds(g, LANES)] & 7) * N_EMBED
                si = (iidx_v[pl.ds(g, LANES)] & 7) * N_EMBED
                acc = mu_v[pl.ds(g, LANES)] + 2.0 * b_v[pl.ds(g, LANES)]
                for e in range(N_EMBED):
                    uv = plsc.load_gather(urows_v, [rows, su + e])
                    iv = plsc.load_gather(irows_v, [rows, si + e])
                    acc = acc + uv * iv
                out_v[pl.ds(g, LANES)] = acc

        pltpu.sync_copy(out_v, out_hbm.at[sl])

    return k(u_idx, i_idx, mu, bias, W_user, W_item)


def kernel(x, W_user, W_item):
    u_idx = x[:, 0].astype(jnp.int32)
    i_idx = x[:, 1].astype(jnp.int32)
    mu = x[:, 2]
    bias = x[:, 3]
    wu = W_user.reshape(W_user.shape[0] // ROWS_PER_LINE, 128)
    wi = W_item.reshape(W_item.shape[0] // ROWS_PER_LINE, 128)
    return _sc_embed_dot(u_idx, i_idx, mu, bias, wu, wi)
